# Initial kernel scaffold; baseline (speedup 1.0000x reference)
import jax
import jax.numpy as jnp
from jax.experimental import pallas as pl


def kernel(x, edge_index, W1, b1, W2, b2):
    raise NotImplementedError("write your pallas kernel here")


# speedup vs baseline: 24.5088x; 24.5088x over previous
import functools

import jax
import jax.numpy as jnp
from jax import lax
from jax.experimental import pallas as pl
from jax.experimental.pallas import tpu as pltpu
from jax.experimental.pallas import tpu_sc as plsc

N = 10000
D = 128
E = 320000
NPAD = 10240
NT = 32
TPW = 79 * 128
EPAD = NT * TPW
KB = 79
STRIDE = NPAD // 16
BR = 1024
GRID = NPAD // BR

_mesh = plsc.VectorSubcoreMesh(core_axis_name="c", subcore_axis_name="s")
_sc_params = pltpu.CompilerParams(needs_layout_passes=False)


@functools.partial(
    pl.kernel,
    out_type=jax.ShapeDtypeStruct((NT, NPAD), jnp.float32),
    mesh=_mesh,
    scratch_types=[
        pltpu.VMEM((KB, 128), jnp.int32),
        pltpu.VMEM((NPAD,), jnp.float32),
        pltpu.SemaphoreType.DMA,
    ],
    compiler_params=_sc_params,
)
def _deg_kernel(dst3, zvec, degp, dst_v, hist, sem):
    cid = lax.axis_index("c")
    sid = lax.axis_index("s")
    wid = cid * 16 + sid
    pltpu.async_copy(dst3.at[wid], dst_v, sem).wait()
    pltpu.sync_copy(zvec, hist)
    ones = jnp.ones((16,), jnp.float32)

    def body(t, carry):
        r = t >> 3
        c = (t & 7) * 16
        d16 = dst_v[r, pl.ds(c, 16)]
        plsc.addupdate_scatter(hist, [d16], ones)
        return carry

    lax.fori_loop(0, KB * 8, body, 0)
    pltpu.sync_copy(hist, degp.at[wid])


def _mm1_body(x_ref, w_ref, degp_ref, hp_ref):
    deg = jnp.sum(degp_ref[...], axis=0) + 1.0
    dinv = lax.rsqrt(deg)
    h = jnp.dot(x_ref[...], w_ref[...], preferred_element_type=jnp.float32)
    hp_ref[...] = h * dinv[:, None]


def _mm1(x_pad, W1, degp):
    return pl.pallas_call(
        _mm1_body,
        grid=(GRID,),
        in_specs=[
            pl.BlockSpec((BR, D), lambda i: (i, 0)),
            pl.BlockSpec((D, D), lambda i: (0, 0)),
            pl.BlockSpec((NT, BR), lambda i: (0, i)),
        ],
        out_specs=pl.BlockSpec((BR, D), lambda i: (i, 0)),
        out_shape=jax.ShapeDtypeStruct((NPAD, D), jnp.float32),
    )(x_pad, W1, degp)


@functools.partial(
    pl.kernel,
    out_type=jax.ShapeDtypeStruct((2, NPAD, D), jnp.float32),
    mesh=_mesh,
    scratch_types=[
        pltpu.VMEM((KB, 128), jnp.int32),
        pltpu.VMEM((KB, 128), jnp.int32),
        pltpu.VMEM((128, D), jnp.float32),
        pltpu.VMEM_SHARED((NPAD, D), jnp.float32),
        pltpu.SemaphoreType.DMA,
        pltpu.SemaphoreType.DMA,
    ],
    compiler_params=_sc_params,
)
def _agg_kernel(hp, src3, dst3, zeros_h, accp, src_v, dst_v, rows, acc_sh,
                sem1, sem2):
    cid = lax.axis_index("c")
    sid = lax.axis_index("s")
    wid = cid * 16 + sid
    stripe = sid * STRIDE

    @pl.when(cid == 0)
    def _():
        pltpu.sync_copy(hp.at[pl.ds(stripe, STRIDE)],
                        acc_sh.at[pl.ds(stripe, STRIDE)])

    @pl.when(cid == 1)
    def _():
        pltpu.sync_copy(zeros_h.at[pl.ds(stripe, STRIDE)],
                        acc_sh.at[pl.ds(stripe, STRIDE)])

    cp_s = pltpu.async_copy(src3.at[wid], src_v, sem1)
    cp_d = pltpu.async_copy(dst3.at[wid], dst_v, sem2)
    cp_s.wait()
    cp_d.wait()
    plsc.subcore_barrier()

    def body(j, carry):
        pltpu.async_copy(hp.at[src_v.at[j]], rows, sem1).wait()
        pltpu.sync_copy(rows, acc_sh.at[dst_v.at[j]], add=True)
        return carry

    lax.fori_loop(0, KB, body, 0)
    plsc.subcore_barrier()
    pltpu.sync_copy(acc_sh.at[pl.ds(stripe, STRIDE)],
                    accp.at[cid, pl.ds(stripe, STRIDE)])


def _mid_body(accp_ref, degp_ref, b1_ref, w2_ref, gp_ref):
    i = pl.program_id(0)
    deg = jnp.sum(degp_ref[...], axis=0) + 1.0
    dinv = lax.rsqrt(deg)
    tot = accp_ref[0] + accp_ref[1]
    out1 = tot * dinv[:, None] + b1_ref[...][None, :]
    r = jnp.maximum(out1, 0.0)
    g = jnp.sum(r * w2_ref[...][None, :], axis=1)
    gpv = dinv * g
    rowid = lax.broadcasted_iota(jnp.int32, (BR, 1), 0)[:, 0] + i * BR
    gp_ref[...] = jnp.where(rowid < N, gpv, 0.0)


def _mid(accp, degp, b1, w2row):
    return pl.pallas_call(
        _mid_body,
        grid=(GRID,),
        in_specs=[
            pl.BlockSpec((2, BR, D), lambda i: (0, i, 0)),
            pl.BlockSpec((NT, BR), lambda i: (0, i)),
            pl.BlockSpec((D,), lambda i: (0,)),
            pl.BlockSpec((D,), lambda i: (0,)),
        ],
        out_specs=pl.BlockSpec((BR,), lambda i: (i,)),
        out_shape=jax.ShapeDtypeStruct((NPAD,), jnp.float32),
    )(accp, degp, b1, w2row)


@functools.partial(
    pl.kernel,
    out_type=jax.ShapeDtypeStruct((NT, NPAD), jnp.float32),
    mesh=_mesh,
    scratch_types=[
        pltpu.VMEM((NPAD,), jnp.float32),
        pltpu.VMEM((NPAD,), jnp.float32),
        pltpu.VMEM((KB, 128), jnp.int32),
        pltpu.VMEM((KB, 128), jnp.int32),
        pltpu.SemaphoreType.DMA,
    ],
    compiler_params=_sc_params,
)
def _agg2_kernel(gp, src3, dst3, zvec, acc2p, gp_v, acc2, src_v, dst_v, sem):
    cid = lax.axis_index("c")
    sid = lax.axis_index("s")
    wid = cid * 16 + sid
    pltpu.async_copy(src3.at[wid], src_v, sem).wait()
    pltpu.async_copy(dst3.at[wid], dst_v, sem).wait()
    pltpu.sync_copy(gp, gp_v)
    pltpu.sync_copy(zvec, acc2)

    def body(t, carry):
        r = t >> 3
        c = (t & 7) * 16
        s16 = src_v[r, pl.ds(c, 16)]
        d16 = dst_v[r, pl.ds(c, 16)]
        v = plsc.load_gather(gp_v, [s16])
        plsc.addupdate_scatter(acc2, [d16], v)
        return carry

    lax.fori_loop(0, KB * 8, body, 0)
    pltpu.sync_copy(acc2, acc2p.at[wid])


def _fin_body(acc2p_ref, gp_ref, degp_ref, b2_ref, out_ref):
    deg = jnp.sum(degp_ref[...], axis=0) + 1.0
    dinv = lax.rsqrt(deg)
    z = dinv * (jnp.sum(acc2p_ref[...], axis=0) + gp_ref[...]) + b2_ref[0]
    out_ref[...] = jax.nn.sigmoid(z)


def _fin(acc2p, gp, degp, b2):
    return pl.pallas_call(
        _fin_body,
        grid=(GRID,),
        in_specs=[
            pl.BlockSpec((NT, BR), lambda i: (0, i)),
            pl.BlockSpec((BR,), lambda i: (i,)),
            pl.BlockSpec((NT, BR), lambda i: (0, i)),
            pl.BlockSpec(memory_space=pltpu.SMEM),
        ],
        out_specs=pl.BlockSpec((BR,), lambda i: (i,)),
        out_shape=jax.ShapeDtypeStruct((NPAD,), jnp.float32),
    )(acc2p, gp, degp, b2)


def kernel(x, edge_index, W1, b1, W2, b2):
    ei = edge_index.astype(jnp.int32)
    src = jnp.concatenate(
        [ei[0], jnp.full((EPAD - E,), N, jnp.int32)]).reshape(NT, KB, 128)
    dst = jnp.concatenate(
        [ei[1], jnp.full((EPAD - E,), N, jnp.int32)]).reshape(NT, KB, 128)
    x_pad = jnp.pad(x, ((0, NPAD - N), (0, 0)))
    zeros_h = jnp.zeros((NPAD, D), jnp.float32)
    zvec = jnp.zeros((NPAD,), jnp.float32)

    degp = _deg_kernel(dst, zvec)
    hp = _mm1(x_pad, W1, degp)
    accp = _agg_kernel(hp, src, dst, zeros_h)
    gp = _mid(accp, degp, b1, W2[:, 0])
    acc2p = _agg2_kernel(gp, src, dst, zvec)
    out = _fin(acc2p, gp, degp, b2)
    return out[:N].reshape(N, 1)


# speedup vs baseline: 26.8161x; 1.0941x over previous
import functools

import jax
import jax.numpy as jnp
from jax import lax
from jax.experimental import pallas as pl
from jax.experimental.pallas import tpu as pltpu
from jax.experimental.pallas import tpu_sc as plsc

N = 10000
D = 128
E = 320000
NPAD = 10240
NT = 32
TPW = 79 * 128
EPAD = NT * TPW
BLK = 128
KB = TPW // BLK
DH = D // 2
STRIDE = NPAD // 16
BR = 1024
GRID = NPAD // BR

_mesh = plsc.VectorSubcoreMesh(core_axis_name="c", subcore_axis_name="s")
_sc_params = pltpu.CompilerParams(needs_layout_passes=False,
                                 use_tc_tiling_on_sc=False)


@functools.partial(
    pl.kernel,
    out_type=jax.ShapeDtypeStruct((NT, NPAD), jnp.float32),
    mesh=_mesh,
    scratch_types=[
        pltpu.VMEM((KB, BLK), jnp.int32),
        pltpu.VMEM((NPAD,), jnp.float32),
        pltpu.SemaphoreType.DMA,
    ],
    compiler_params=_sc_params,
)
def _deg_kernel(dst3, zvec, degp, dst_v, hist, sem):
    cid = lax.axis_index("c")
    sid = lax.axis_index("s")
    wid = cid * 16 + sid
    pltpu.async_copy(dst3.at[wid], dst_v, sem).wait()
    pltpu.sync_copy(zvec, hist)
    ones = jnp.ones((16,), jnp.float32)

    def body(t, carry):
        r = t >> 3
        c = (t & 7) * 16
        d16 = dst_v[r, pl.ds(c, 16)]
        plsc.addupdate_scatter(hist, [d16], ones)
        return carry

    lax.fori_loop(0, KB * 8, body, 0)
    pltpu.sync_copy(hist, degp.at[wid])


def _mm1_body(x_ref, w_ref, degp_ref, hp_ref):
    deg = jnp.sum(degp_ref[...], axis=0) + 1.0
    dinv = lax.rsqrt(deg)
    h = jnp.dot(x_ref[...], w_ref[...], preferred_element_type=jnp.float32)
    hp = h * dinv[:, None]
    hp_ref[0] = hp[:, :DH]
    hp_ref[1] = hp[:, DH:]


def _mm1(x_pad, W1, degp):
    return pl.pallas_call(
        _mm1_body,
        grid=(GRID,),
        in_specs=[
            pl.BlockSpec((BR, D), lambda i: (i, 0)),
            pl.BlockSpec((D, D), lambda i: (0, 0)),
            pl.BlockSpec((NT, BR), lambda i: (0, i)),
        ],
        out_specs=pl.BlockSpec((2, BR, DH), lambda i: (0, i, 0)),
        out_shape=jax.ShapeDtypeStruct((2, NPAD, DH), jnp.float32),
    )(x_pad, W1, degp)


@functools.partial(
    pl.kernel,
    out_type=jax.ShapeDtypeStruct((2, 2, NPAD, DH), jnp.float32),
    mesh=_mesh,
    scratch_types=[
        pltpu.VMEM((KB, BLK), jnp.int32),
        pltpu.VMEM((KB, BLK), jnp.int32),
        pltpu.VMEM((2 * BLK, DH), jnp.float32),
        pltpu.VMEM_SHARED((NPAD, DH), jnp.float32),
        pltpu.SemaphoreType.DMA,
        pltpu.SemaphoreType.DMA,
    ],
    compiler_params=_sc_params,
)
def _agg_kernel(hp2, src3, dst3, zeros_h, accp, src_v, dst_v, rows2,
                acc_sh, gsem, ssem):
    cid = lax.axis_index("c")
    sid = lax.axis_index("s")
    wid = cid * 16 + sid
    stripe = sid * STRIDE

    cp_s = pltpu.async_copy(src3.at[wid], src_v, gsem)
    cp_d = pltpu.async_copy(dst3.at[wid], dst_v, ssem)
    cp_s.wait()
    cp_d.wait()

    def dpass(d, carry):
        @pl.when(cid == 0)
        def _():
            pltpu.sync_copy(hp2.at[d, pl.ds(stripe, STRIDE)],
                            acc_sh.at[pl.ds(stripe, STRIDE)])

        @pl.when(cid == 1)
        def _():
            pltpu.sync_copy(zeros_h.at[pl.ds(stripe, STRIDE)],
                            acc_sh.at[pl.ds(stripe, STRIDE)])

        plsc.subcore_barrier()

        def ga(j):
            return pltpu.make_async_copy(
                hp2.at[d].at[src_v.at[j]],
                rows2.at[pl.ds((j % 2) * BLK, BLK)], gsem)

        def sc(j):
            return pltpu.make_async_copy(
                rows2.at[pl.ds((j % 2) * BLK, BLK)],
                acc_sh.at[dst_v.at[j]], ssem)

        def body(j, carry2):
            @pl.when((j >= 2) & (j - 2 < KB))
            def _():
                sc(j - 2).wait()

            @pl.when(j < KB)
            def _():
                ga(j).start()

            @pl.when((j >= 1) & (j - 1 < KB))
            def _():
                ga(j - 1).wait()
                sc(j - 1).start(add=True)
            return carry2

        lax.fori_loop(0, KB + 2, body, 0)
        plsc.subcore_barrier()
        pltpu.sync_copy(acc_sh.at[pl.ds(stripe, STRIDE)],
                        accp.at[cid, d, pl.ds(stripe, STRIDE)])
        return carry

    lax.fori_loop(0, 2, dpass, 0)


def _mid_body(accp_ref, degp_ref, b1_ref, w2_ref, gp_ref):
    i = pl.program_id(0)
    deg = jnp.sum(degp_ref[...], axis=0) + 1.0
    dinv = lax.rsqrt(deg)
    b1 = b1_ref[...]
    w2 = w2_ref[...]
    g = jnp.zeros((BR,), jnp.float32)
    for d in range(2):
        tot = accp_ref[0, d] + accp_ref[1, d]
        out1 = tot * dinv[:, None] + b1[d * DH:(d + 1) * DH][None, :]
        r = jnp.maximum(out1, 0.0)
        g = g + jnp.sum(r * w2[d * DH:(d + 1) * DH][None, :], axis=1)
    gpv = dinv * g
    rowid = lax.broadcasted_iota(jnp.int32, (BR, 1), 0)[:, 0] + i * BR
    gp_ref[...] = jnp.where(rowid < N, gpv, 0.0)


def _mid(accp, degp, b1, w2row):
    return pl.pallas_call(
        _mid_body,
        grid=(GRID,),
        in_specs=[
            pl.BlockSpec((2, 2, BR, DH), lambda i: (0, 0, i, 0)),
            pl.BlockSpec((NT, BR), lambda i: (0, i)),
            pl.BlockSpec((D,), lambda i: (0,)),
            pl.BlockSpec((D,), lambda i: (0,)),
        ],
        out_specs=pl.BlockSpec((BR,), lambda i: (i,)),
        out_shape=jax.ShapeDtypeStruct((NPAD,), jnp.float32),
    )(accp, degp, b1, w2row)


@functools.partial(
    pl.kernel,
    out_type=jax.ShapeDtypeStruct((NT, NPAD), jnp.float32),
    mesh=_mesh,
    scratch_types=[
        pltpu.VMEM((NPAD,), jnp.float32),
        pltpu.VMEM((NPAD,), jnp.float32),
        pltpu.VMEM((KB, BLK), jnp.int32),
        pltpu.VMEM((KB, BLK), jnp.int32),
        pltpu.SemaphoreType.DMA,
    ],
    compiler_params=_sc_params,
)
def _agg2_kernel(gp, src3, dst3, zvec, acc2p, gp_v, acc2, src_v, dst_v, sem):
    cid = lax.axis_index("c")
    sid = lax.axis_index("s")
    wid = cid * 16 + sid
    pltpu.async_copy(src3.at[wid], src_v, sem).wait()
    pltpu.async_copy(dst3.at[wid], dst_v, sem).wait()
    pltpu.sync_copy(gp, gp_v)
    pltpu.sync_copy(zvec, acc2)

    def body(t, carry):
        r = t >> 3
        c = (t & 7) * 16
        s16 = src_v[r, pl.ds(c, 16)]
        d16 = dst_v[r, pl.ds(c, 16)]
        v = plsc.load_gather(gp_v, [s16])
        plsc.addupdate_scatter(acc2, [d16], v)
        return carry

    lax.fori_loop(0, KB * 8, body, 0)
    pltpu.sync_copy(acc2, acc2p.at[wid])


def _fin_body(acc2p_ref, gp_ref, degp_ref, b2_ref, out_ref):
    deg = jnp.sum(degp_ref[...], axis=0) + 1.0
    dinv = lax.rsqrt(deg)
    z = dinv * (jnp.sum(acc2p_ref[...], axis=0) + gp_ref[...]) + b2_ref[0]
    out_ref[...] = jax.nn.sigmoid(z)


def _fin(acc2p, gp, degp, b2):
    return pl.pallas_call(
        _fin_body,
        grid=(GRID,),
        in_specs=[
            pl.BlockSpec((NT, BR), lambda i: (0, i)),
            pl.BlockSpec((BR,), lambda i: (i,)),
            pl.BlockSpec((NT, BR), lambda i: (0, i)),
            pl.BlockSpec(memory_space=pltpu.SMEM),
        ],
        out_specs=pl.BlockSpec((BR,), lambda i: (i,)),
        out_shape=jax.ShapeDtypeStruct((NPAD,), jnp.float32),
    )(acc2p, gp, degp, b2)


def kernel(x, edge_index, W1, b1, W2, b2):
    ei = edge_index.astype(jnp.int32)
    src = jnp.concatenate(
        [ei[0], jnp.full((EPAD - E,), N, jnp.int32)]).reshape(NT, KB, BLK)
    dst = jnp.concatenate(
        [ei[1], jnp.full((EPAD - E,), N, jnp.int32)]).reshape(NT, KB, BLK)
    x_pad = jnp.pad(x, ((0, NPAD - N), (0, 0)))
    zeros_h = jnp.zeros((NPAD, DH), jnp.float32)
    zvec = jnp.zeros((NPAD,), jnp.float32)

    degp = _deg_kernel(dst, zvec)
    hp = _mm1(x_pad, W1, degp)
    accp = _agg_kernel(hp, src, dst, zeros_h)
    gp = _mid(accp, degp, b1, W2[:, 0])
    acc2p = _agg2_kernel(gp, src, dst, zvec)
    out = _fin(acc2p, gp, degp, b2)
    return out[:N].reshape(N, 1)
